# baseline (speedup 1.0000x reference)
import jax
import jax.numpy as jnp
from jax.experimental import pallas as pl


def _body(f_ref, m_ref, t_ref, o_ref):
    o_ref[...] = jnp.where(m_ref[...], t_ref[...], f_ref[...])


def kernel(features, mask, mask_token):
    B, S, D = features.shape
    N = B * S
    R = 1024
    f2 = features.reshape(N, D)
    m2 = mask.reshape(N, 1)
    t2 = mask_token.reshape(1, D)
    grid = (N // R,)
    out = pl.pallas_call(
        _body,
        grid=grid,
        in_specs=[
            pl.BlockSpec((R, D), lambda i: (i, 0)),
            pl.BlockSpec((R, 1), lambda i: (i, 0)),
            pl.BlockSpec((1, D), lambda i: (0, 0)),
        ],
        out_specs=pl.BlockSpec((R, D), lambda i: (i, 0)),
        out_shape=jax.ShapeDtypeStruct((N, D), features.dtype),
    )(f2, m2, t2)
    return out.reshape(B, S, D)


# speedup vs baseline: 1.0082x; 1.0082x over previous
import functools

import jax
import jax.numpy as jnp
from jax.experimental import pallas as pl
from jax.experimental.pallas import tpu as pltpu


def _body(N, D, R, K, f_ref, m_ref, t_ref, o_ref, in_buf, out_buf, in_sem, out_sem):
    steps = N // R

    def in_dma(chunk, slot):
        return pltpu.make_async_copy(
            f_ref.at[pl.ds(chunk * R, R), :], in_buf.at[slot], in_sem.at[slot]
        )

    def out_dma(chunk, slot):
        return pltpu.make_async_copy(
            out_buf.at[slot], o_ref.at[pl.ds(chunk * R, R), :], out_sem.at[slot]
        )

    for j in range(K):
        in_dma(j, j).start()

    def step(i, carry):
        slot = jax.lax.rem(i, K)
        in_dma(i, slot).wait()

        @pl.when(i >= K)
        def _():
            out_dma(i - K, slot).wait()

        m = m_ref[pl.ds(i * R, R), :]
        out_buf[slot] = jnp.where(m, t_ref[...], in_buf[slot])
        out_dma(i, slot).start()

        @pl.when(i + K < steps)
        def _():
            in_dma(i + K, slot).start()

        return carry

    jax.lax.fori_loop(0, steps, step, 0)
    for j in range(steps - K, steps):
        out_dma(j, j % K).wait()


def kernel(features, mask, mask_token):
    B, S, D = features.shape
    N = B * S
    R = 512
    K = 6
    f2 = features.reshape(N, D)
    m2 = mask.reshape(N, 1)
    t2 = mask_token.reshape(1, D)
    out = pl.pallas_call(
        functools.partial(_body, N, D, R, K),
        in_specs=[
            pl.BlockSpec(memory_space=pl.ANY),
            pl.BlockSpec(memory_space=pltpu.VMEM),
            pl.BlockSpec(memory_space=pltpu.VMEM),
        ],
        out_specs=pl.BlockSpec(memory_space=pl.ANY),
        out_shape=jax.ShapeDtypeStruct((N, D), features.dtype),
        scratch_shapes=[
            pltpu.VMEM((K, R, D), features.dtype),
            pltpu.VMEM((K, R, D), features.dtype),
            pltpu.SemaphoreType.DMA((K,)),
            pltpu.SemaphoreType.DMA((K,)),
        ],
    )(f2, m2, t2)
    return out.reshape(B, S, D)


# speedup vs baseline: 1.1412x; 1.1319x over previous
import jax
import jax.numpy as jnp
from jax.experimental import pallas as pl


def _body(f_ref, m_ref, t_ref, o_ref):
    R = f_ref.shape[0]
    m = m_ref[0].astype(jnp.int32).reshape(R, 1) != 0
    o_ref[...] = jnp.where(m, t_ref[...], f_ref[...])


def kernel(features, mask, mask_token):
    B, S, D = features.shape
    N = B * S
    R = 1024
    f2 = features.reshape(N, D)
    m3 = mask.reshape(N // R, 1, R)
    t2 = mask_token.reshape(1, D)
    grid = (N // R,)
    out = pl.pallas_call(
        _body,
        grid=grid,
        in_specs=[
            pl.BlockSpec((R, D), lambda i: (i, 0)),
            pl.BlockSpec((1, 1, R), lambda i: (i, 0, 0)),
            pl.BlockSpec((1, D), lambda i: (0, 0)),
        ],
        out_specs=pl.BlockSpec((R, D), lambda i: (i, 0)),
        out_shape=jax.ShapeDtypeStruct((N, D), features.dtype),
    )(f2, m3, t2)
    return out.reshape(B, S, D)
